# baseline (speedup 1.0000x reference)
import functools

import jax
import jax.numpy as jnp
from jax import lax
from jax.experimental import pallas as pl
from jax.experimental.pallas import tpu as pltpu
from jax.experimental.pallas import tpu_sc as plsc

_VOCAB = 100000
_D = 16
_B = 16384
_NC = 2
_NS = 16
_NW = _NC * _NS
_BPW = _B // _NW
_REC = 1 + _D


@functools.partial(
    pl.kernel,
    mesh=plsc.VectorSubcoreMesh(core_axis_name="c", subcore_axis_name="s"),
    out_type=jax.ShapeDtypeStruct((_REC, _B), jnp.float32),
    compiler_params=pltpu.CompilerParams(use_tc_tiling_on_sc=False),
    scratch_types=[
        pltpu.VMEM((_BPW,), jnp.int32),
        pltpu.VMEM((_D * _BPW,), jnp.int32),
        pltpu.VMEM((_REC * _BPW,), jnp.float32),
        pltpu.SemaphoreType.DMA,
        pltpu.SemaphoreType.DMA,
    ],
)
def _emb_concat(pitch_hbm, idx_hbm, tab_t_hbm, out_hbm, idx_v, idx2_v,
                out_v, sem, osem):
    wid = lax.axis_index("s") * _NC + lax.axis_index("c")
    base = wid * _BPW
    pltpu.sync_copy(idx_hbm.at[pl.ds(base, _BPW)], idx_v)

    def body(k, _):
        v = idx_v[pl.ds(k * 16, 16)]
        for c in range(_D):
            idx2_v[pl.ds(c * _BPW + k * 16, 16)] = v + c * _VOCAB
        return 0

    lax.fori_loop(0, _BPW // 16, body, 0)
    gather = pltpu.async_copy(tab_t_hbm.at[idx2_v],
                              out_v.at[pl.ds(_BPW, _D * _BPW)], sem)
    pltpu.sync_copy(pitch_hbm.at[pl.ds(base, _BPW)],
                    out_v.at[pl.ds(0, _BPW)])
    w0 = pltpu.async_copy(out_v.at[pl.ds(0, _BPW)],
                          out_hbm.at[0, pl.ds(base, _BPW)], osem)
    gather.wait()
    writes = [w0]
    for p in range(1, _REC):
        writes.append(
            pltpu.async_copy(out_v.at[pl.ds(p * _BPW, _BPW)],
                             out_hbm.at[p, pl.ds(base, _BPW)], osem))
    for w in writes:
        w.wait()


def kernel(pitch, timbre_id, table):
    return _emb_concat(pitch, timbre_id, table.T.reshape(-1)).T
